# Initial kernel scaffold; baseline (speedup 1.0000x reference)
import jax
import jax.numpy as jnp
from jax.experimental import pallas as pl


def kernel(fnode, fmess, agraph, bgraph, atom_scope, bond_scope, W_in, b_in, W_msg, b_msg, W_ih, b_ih, W_hh, b_hh, W_out, b_out):
    raise NotImplementedError("write your pallas kernel here")


# speedup vs baseline: 8.5873x; 8.5873x over previous
import jax
import jax.numpy as jnp
from jax import lax
from jax.experimental import pallas as pl
from jax.experimental.pallas import tpu as pltpu
from jax.experimental.pallas import tpu_sc as plsc

_N = 10000
_E = 320000
_H = 128
_B = 32

_NC = 2
_NS = 16
_NW = _NC * _NS
_EPT = _E // _NW
_CH = 80
_NCH = _EPT // _CH
_RPS = 624
_REM = _N - _NS * _RPS
_ZR = 16

_BLK = 1000
_GRID = _N // _BLK


def _enc_body(fnode_ref, w_in_ref, b_in_ref, w_msg_ref, b_msg_ref,
              h_ref, p_ref):
    h = jnp.maximum(
        jnp.dot(fnode_ref[...], w_in_ref[...],
                preferred_element_type=jnp.float32) + b_in_ref[...], 0.0)
    h_ref[...] = h
    p_ref[...] = jnp.maximum(
        jnp.dot(h, w_msg_ref[...],
                preferred_element_type=jnp.float32) + b_msg_ref[...], 0.0)


def _sc_body(p_hbm, u_hbm, v_hbm, agg_hbm,
             u0, u1, v0, v1, rows0, rows1, zbuf, shared, sem0, sem1):
    c = lax.axis_index("c")
    s = lax.axis_index("s")
    wid = s * _NC + c
    ebase = wid * _EPT

    zero16 = jnp.zeros((16,), jnp.float32)
    for i in range(_ZR):
        for j in range(_H // 16):
            zbuf[i, pl.ds(j * 16, 16)] = zero16
    for t in range(_RPS // _ZR):
        pltpu.sync_copy(zbuf, shared.at[pl.ds(s * _RPS + t * _ZR, _ZR)])

    @pl.when(s == 0)
    def _zero_rem():
        pltpu.sync_copy(zbuf, shared.at[pl.ds(_NS * _RPS, _REM)])

    plsc.subcore_barrier()

    def _stage(ck, u_b, v_b):
        pltpu.sync_copy(u_hbm.at[pl.ds(ebase + ck * _CH, _CH)], u_b)
        pltpu.sync_copy(v_hbm.at[pl.ds(ebase + ck * _CH, _CH)], v_b)

    _stage(0, u0, v0)
    pltpu.async_copy(p_hbm.at[u0], rows0, sem0)

    @pl.loop(0, (_NCH - 1) // 2)
    def _chunks(g):
        c0 = 2 * g
        _stage(c0 + 1, u1, v1)
        pltpu.async_copy(p_hbm.at[u1], rows1, sem1)
        pltpu.make_async_copy(p_hbm.at[u0], rows0, sem0).wait()
        pltpu.sync_copy(rows0, shared.at[v0], add=True)
        _stage(c0 + 2, u0, v0)
        pltpu.async_copy(p_hbm.at[u0], rows0, sem0)
        pltpu.make_async_copy(p_hbm.at[u1], rows1, sem1).wait()
        pltpu.sync_copy(rows1, shared.at[v1], add=True)

    pltpu.make_async_copy(p_hbm.at[u0], rows0, sem0).wait()
    pltpu.sync_copy(rows0, shared.at[v0], add=True)

    plsc.subcore_barrier()
    pltpu.sync_copy(shared.at[pl.ds(s * _RPS, _RPS)],
                    agg_hbm.at[c, pl.ds(s * _RPS, _RPS)])

    @pl.when(s == 0)
    def _copy_rem():
        pltpu.sync_copy(shared.at[pl.ds(_NS * _RPS, _REM)],
                        agg_hbm.at[c, pl.ds(_NS * _RPS, _REM)])


def _gru_pool_body(h_ref, agg_ref, st_ref, ln_ref,
                   w_ih_ref, b_ih_ref, w_hh_ref, b_hh_ref,
                   w_out_ref, b_out_ref, out_ref, acc_ref):
    k = pl.program_id(0)
    h = h_ref[...]
    agg = agg_ref[0] + agg_ref[1]
    gi = jnp.dot(agg, w_ih_ref[...],
                 preferred_element_type=jnp.float32) + b_ih_ref[...]
    gh = jnp.dot(h, w_hh_ref[...],
                 preferred_element_type=jnp.float32) + b_hh_ref[...]
    r = jax.nn.sigmoid(gi[:, :_H] + gh[:, :_H])
    z = jax.nn.sigmoid(gi[:, _H:2 * _H] + gh[:, _H:2 * _H])
    n = jnp.tanh(gi[:, 2 * _H:] + r * gh[:, 2 * _H:])
    h2 = jnp.maximum((1.0 - z) * n + z * h, 0.0)

    ids = k * _BLK + lax.broadcasted_iota(jnp.int32, (_B, _BLK), 1)
    st = st_ref[...]
    ln = ln_ref[...]
    mask = (ids >= st) & (ids < st + ln)

    @pl.when(k == 0)
    def _init():
        acc_ref[...] = jnp.zeros_like(acc_ref)

    acc_ref[...] += jnp.dot(mask.astype(jnp.float32), h2,
                            preferred_element_type=jnp.float32)

    @pl.when(k == pl.num_programs(0) - 1)
    def _fin():
        denom = jnp.maximum(ln, 1).astype(jnp.float32)
        g = jnp.where(ln > 0, acc_ref[...] / denom, 0.0)
        out_ref[...] = jnp.dot(g, w_out_ref[...],
                               preferred_element_type=jnp.float32) \
            + b_out_ref[...]


def kernel(fnode, fmess, agraph, bgraph, atom_scope, bond_scope,
           W_in, b_in, W_msg, b_msg, W_ih, b_ih, W_hh, b_hh, W_out, b_out):
    h, p = pl.pallas_call(
        _enc_body,
        grid=(_GRID,),
        in_specs=[
            pl.BlockSpec((_BLK, _H), lambda k: (k, 0)),
            pl.BlockSpec((_H, _H), lambda k: (0, 0)),
            pl.BlockSpec((1, _H), lambda k: (0, 0)),
            pl.BlockSpec((_H, _H), lambda k: (0, 0)),
            pl.BlockSpec((1, _H), lambda k: (0, 0)),
        ],
        out_specs=[
            pl.BlockSpec((_BLK, _H), lambda k: (k, 0)),
            pl.BlockSpec((_BLK, _H), lambda k: (k, 0)),
        ],
        out_shape=[
            jax.ShapeDtypeStruct((_N, _H), jnp.float32),
            jax.ShapeDtypeStruct((_N, _H), jnp.float32),
        ],
    )(fnode, W_in, b_in.reshape(1, _H), W_msg, b_msg.reshape(1, _H))

    u1d = fmess[:, 0]
    v1d = fmess[:, 1]
    mesh = plsc.VectorSubcoreMesh(core_axis_name="c", subcore_axis_name="s",
                                  num_cores=_NC, num_subcores=_NS)
    agg2 = pl.kernel(
        _sc_body,
        out_type=jax.ShapeDtypeStruct((_NC, _N, _H), jnp.float32),
        mesh=mesh,
        scratch_types=[
            pltpu.VMEM((_CH,), jnp.int32),
            pltpu.VMEM((_CH,), jnp.int32),
            pltpu.VMEM((_CH,), jnp.int32),
            pltpu.VMEM((_CH,), jnp.int32),
            pltpu.VMEM((_CH, _H), jnp.float32),
            pltpu.VMEM((_CH, _H), jnp.float32),
            pltpu.VMEM((_ZR, _H), jnp.float32),
            pltpu.VMEM_SHARED((_N, _H), jnp.float32),
            pltpu.SemaphoreType.DMA,
            pltpu.SemaphoreType.DMA,
        ],
    )(p, u1d, v1d)

    st = atom_scope[:, 0, 0]
    ln = atom_scope[:, -1, 0] + atom_scope[:, -1, 1] - st
    out = pl.pallas_call(
        _gru_pool_body,
        grid=(_GRID,),
        in_specs=[
            pl.BlockSpec((_BLK, _H), lambda k: (k, 0)),
            pl.BlockSpec((_NC, _BLK, _H), lambda k: (0, k, 0)),
            pl.BlockSpec((_B, 1), lambda k: (0, 0)),
            pl.BlockSpec((_B, 1), lambda k: (0, 0)),
            pl.BlockSpec((_H, 3 * _H), lambda k: (0, 0)),
            pl.BlockSpec((1, 3 * _H), lambda k: (0, 0)),
            pl.BlockSpec((_H, 3 * _H), lambda k: (0, 0)),
            pl.BlockSpec((1, 3 * _H), lambda k: (0, 0)),
            pl.BlockSpec((_H, _H), lambda k: (0, 0)),
            pl.BlockSpec((1, _H), lambda k: (0, 0)),
        ],
        out_specs=pl.BlockSpec((_B, _H), lambda k: (0, 0)),
        out_shape=jax.ShapeDtypeStruct((_B, _H), jnp.float32),
        scratch_shapes=[pltpu.VMEM((_B, _H), jnp.float32)],
    )(h, agg2, st.reshape(_B, 1), ln.reshape(_B, 1),
      W_ih, b_ih.reshape(1, 3 * _H), W_hh, b_hh.reshape(1, 3 * _H),
      W_out, b_out.reshape(1, _H))
    return out
